# Initial kernel scaffold; baseline (speedup 1.0000x reference)
import jax
import jax.numpy as jnp
from jax.experimental import pallas as pl


def kernel(voxels, embed1, W1, b1, embed2, W2, b2, W3, b3, W4, b4):
    raise NotImplementedError("write your pallas kernel here")


# speedup vs baseline: 70.8789x; 70.8789x over previous
import functools

import jax
import jax.numpy as jnp
from jax import lax
from jax.experimental import pallas as pl
from jax.experimental.pallas import tpu as pltpu
from jax.experimental.pallas import tpu_sc as plsc

NC = 2
NS = 16
NW = NC * NS
NVOX = 4913
NPAD = 4928
NIT = NPAD // 16
HSZ = 17 * 256


def _sc_histogram(vflat, B):
    S = B // NW
    assert B % NW == 0 and S % 2 == 0
    mesh = plsc.VectorSubcoreMesh(core_axis_name="c", subcore_axis_name="s")

    @functools.partial(
        pl.kernel,
        mesh=mesh,
        out_type=jax.ShapeDtypeStruct((B, 4096), jnp.float32),
        compiler_params=pltpu.CompilerParams(needs_layout_passes=False),
        scratch_types=[
            pltpu.VMEM((NPAD,), jnp.int32),
            pltpu.VMEM((NPAD,), jnp.int32),
            pltpu.VMEM((HSZ,), jnp.float32),
            pltpu.VMEM((HSZ,), jnp.float32),
            pltpu.SemaphoreType.DMA,
            pltpu.SemaphoreType.DMA,
            pltpu.SemaphoreType.DMA,
            pltpu.SemaphoreType.DMA,
        ],
    )
    def hist_kernel(vflat_hbm, hout_hbm, vox0, vox1, hist0, hist1,
                    si0, si1, so0, so1):
        wid = lax.axis_index("s") * NC + lax.axis_index("c")
        base = wid * S
        lane = lax.iota(jnp.int32, 16)
        lbase = lane * 256
        trash = lane + 4096
        ones = jnp.ones((16,), jnp.float32)
        zeros = jnp.zeros((16,), jnp.float32)

        def astart(s):
            off = s * NVOX
            return pl.multiple_of((off >> 3) << 3, 8)

        def delta(s):
            return (s * NVOX) & 7

        def do_sample(s, vox, hist):
            d = delta(s)
            def zbody(i, c):
                hist[pl.ds(i * 16, 16)] = zeros
                return c
            lax.fori_loop(0, 256, zbody, 0, unroll=8)
            v = vox[pl.ds(0, 16)]
            idx = jnp.where(lane >= d, lbase + v, trash)
            plsc.addupdate_scatter(hist, [idx], ones)

            def body(i, c):
                vv = vox[pl.ds(i * 16, 16)]
                plsc.addupdate_scatter(hist, [lbase + vv], ones)
                return c
            lax.fori_loop(1, NIT - 1, body, 0, unroll=6)
            v = vox[pl.ds((NIT - 1) * 16, 16)]
            idx = jnp.where(lane <= d, lbase + v, trash)
            plsc.addupdate_scatter(hist, [idx], ones)

        pltpu.async_copy(vflat_hbm.at[pl.ds(astart(base), NPAD)], vox0, si0)

        def pair(k, c):
            s0 = base + 2 * k
            s1 = s0 + 1
            pltpu.async_copy(vflat_hbm.at[pl.ds(astart(s1), NPAD)], vox1, si1)
            pltpu.make_async_copy(
                vflat_hbm.at[pl.ds(astart(s0), NPAD)], vox0, si0).wait()

            @pl.when(k >= 1)
            def _():
                pltpu.make_async_copy(
                    hist0.at[pl.ds(0, 4096)], hout_hbm.at[s0 - 2], so0).wait()

            do_sample(s0, vox0, hist0)
            pltpu.async_copy(hist0.at[pl.ds(0, 4096)], hout_hbm.at[s0], so0)

            @pl.when(k + 1 < S // 2)
            def _():
                pltpu.async_copy(
                    vflat_hbm.at[pl.ds(astart(s0 + 2), NPAD)], vox0, si0)

            pltpu.make_async_copy(
                vflat_hbm.at[pl.ds(astart(s1), NPAD)], vox1, si1).wait()

            @pl.when(k >= 1)
            def _():
                pltpu.make_async_copy(
                    hist1.at[pl.ds(0, 4096)], hout_hbm.at[s1 - 2], so1).wait()

            do_sample(s1, vox1, hist1)
            pltpu.async_copy(hist1.at[pl.ds(0, 4096)], hout_hbm.at[s1], so1)
            return c

        lax.fori_loop(0, S // 2, pair, 0)
        pltpu.make_async_copy(
            hist0.at[pl.ds(0, 4096)], hout_hbm.at[base + S - 2], so0).wait()
        pltpu.make_async_copy(
            hist1.at[pl.ds(0, 4096)], hout_hbm.at[base + S - 1], so1).wait()

    return hist_kernel(vflat)


def _tbl_body(e2_ref, w2_ref, out_ref):
    out_ref[...] = jnp.dot(
        e2_ref[...].astype(jnp.float32), w2_ref[...].astype(jnp.float32),
        preferred_element_type=jnp.float32).astype(jnp.bfloat16)


def _make_tbl(embed2, W2):
    return pl.pallas_call(
        _tbl_body,
        grid=(125,),
        in_specs=[
            pl.BlockSpec((256, 16), lambda p: (0, 0)),
            pl.BlockSpec((16, 128), lambda p: (p, 0)),
        ],
        out_specs=pl.BlockSpec((256, 128), lambda p: (p, 0)),
        out_shape=jax.ShapeDtypeStruct((125 * 256, 128), jnp.bfloat16),
    )(embed2, W2)


def _main_body(h_ref, cv_ref, rg_ref, bl_ref, tbl_ref, e1_ref, w1_ref, b1_ref,
               b2_ref, w3_ref, b3_ref, w4_ref, b4_ref, out_ref, oh_ref):
    hh = h_ref[...]
    counts = hh[:, 0:256]
    for l in range(1, 16):
        counts = counts + hh[:, l * 256:(l + 1) * 256]
    mean = jnp.dot(counts, e1_ref[...],
                   preferred_element_type=jnp.float32) * (1.0 / NVOX)
    hf = jax.nn.relu(jnp.dot(mean, w1_ref[...],
                             preferred_element_type=jnp.float32) + b1_ref[...])

    cvv = cv_ref[...]
    iota = lax.broadcasted_iota(jnp.int32, (cvv.shape[0], 256), 1)
    for p in range(125):
        oh_ref[:, 256 * p:256 * (p + 1)] = (
            cvv[:, p:p + 1] == iota).astype(jnp.bfloat16)
    sp = jax.nn.relu(
        jnp.dot(oh_ref[...], tbl_ref[...],
                preferred_element_type=jnp.float32) + b2_ref[...])

    rgv = rg_ref[...]
    logc = jnp.sum(((rgv >= 8) & (rgv <= 10)).astype(jnp.float32),
                   axis=1, keepdims=True) * (1.0 / 125.0)
    orec = jnp.sum(((rgv >= 14) & (rgv <= 19)).astype(jnp.float32),
                   axis=1, keepdims=True) * (1.0 / 125.0)
    blv = bl_ref[...]
    sol = jnp.sum(((blv > 0) & (blv != 255)).astype(jnp.float32),
                  axis=1, keepdims=True) * (1.0 / 8.0)
    w3 = w3_ref[...]
    kf = jax.nn.relu(logc * w3[0:1, :] + orec * w3[1:2, :] + sol * w3[2:3, :]
                     + b3_ref[...])

    w4 = w4_ref[...]
    out = (jnp.dot(hf, w4[0:64], preferred_element_type=jnp.float32)
           + jnp.dot(sp, w4[64:192], preferred_element_type=jnp.float32)
           + jnp.dot(kf, w4[192:224], preferred_element_type=jnp.float32)
           + b4_ref[...])
    out_ref[...] = jax.nn.relu(out)


def _main(h, center, region, below, tbl, embed1, W1, b1, b2, W3, b3, W4, b4):
    B = h.shape[0]
    BT = 256
    grid = (B // BT,)
    rep = lambda i: (0, 0)
    row = lambda i: (i, 0)
    return pl.pallas_call(
        _main_body,
        grid=grid,
        in_specs=[
            pl.BlockSpec((BT, 4096), row),
            pl.BlockSpec((BT, 125), row),
            pl.BlockSpec((BT, 729), row),
            pl.BlockSpec((BT, 8), row),
            pl.BlockSpec((125 * 256, 128), rep),
            pl.BlockSpec((256, 8), rep),
            pl.BlockSpec((8, 64), rep),
            pl.BlockSpec((1, 64), rep),
            pl.BlockSpec((1, 128), rep),
            pl.BlockSpec((3, 32), rep),
            pl.BlockSpec((1, 32), rep),
            pl.BlockSpec((224, 256), rep),
            pl.BlockSpec((1, 256), rep),
        ],
        out_specs=pl.BlockSpec((BT, 256), row),
        out_shape=jax.ShapeDtypeStruct((B, 256), jnp.float32),
        scratch_shapes=[pltpu.VMEM((BT, 125 * 256), jnp.bfloat16)],
    )(h, center, region, below, tbl, embed1, W1, b1, b2, W3, b3, W4, b4)


def kernel(voxels, embed1, W1, b1, embed2, W2, b2, W3, b3, W4, b4):
    B = voxels.shape[0]
    vflat = voxels.reshape(-1)
    center = voxels[:, 6:11, 6:11, 6:11].reshape(B, 125)
    region = voxels[:, 4:13, 4:13, 4:13].reshape(B, 729)
    below = voxels[:, 8, 0:8, 8]

    h = _sc_histogram(vflat, B)
    tbl = _make_tbl(embed2, W2)
    return _main(h, center, region, below, tbl, embed1,
                 W1, b1.reshape(1, 64), b2.reshape(1, 128),
                 W3, b3.reshape(1, 32), W4, b4.reshape(1, 256))


# speedup vs baseline: 74.0823x; 1.0452x over previous
import functools

import jax
import jax.numpy as jnp
from jax import lax
from jax.experimental import pallas as pl
from jax.experimental.pallas import tpu as pltpu
from jax.experimental.pallas import tpu_sc as plsc

NC = 2
NS = 16
NW = NC * NS
NVOX = 4913
NPAD = 4928
NIT = NPAD // 16
HSZ = 17 * 256


def _sc_histogram(vflat, B):
    S = B // NW
    assert B % NW == 0 and S % 2 == 0
    mesh = plsc.VectorSubcoreMesh(core_axis_name="c", subcore_axis_name="s")

    @functools.partial(
        pl.kernel,
        mesh=mesh,
        out_type=jax.ShapeDtypeStruct((B, 4096), jnp.float32),
        compiler_params=pltpu.CompilerParams(needs_layout_passes=False),
        scratch_types=[
            pltpu.VMEM((NPAD,), jnp.int32),
            pltpu.VMEM((NPAD,), jnp.int32),
            pltpu.VMEM((HSZ,), jnp.float32),
            pltpu.VMEM((HSZ,), jnp.float32),
            pltpu.SemaphoreType.DMA,
            pltpu.SemaphoreType.DMA,
            pltpu.SemaphoreType.DMA,
            pltpu.SemaphoreType.DMA,
        ],
    )
    def hist_kernel(vflat_hbm, hout_hbm, vox0, vox1, hist0, hist1,
                    si0, si1, so0, so1):
        wid = lax.axis_index("s") * NC + lax.axis_index("c")
        base = wid * S
        lane = lax.iota(jnp.int32, 16)
        lbase = lane * 256
        trash = lane + 4096
        ones = jnp.ones((16,), jnp.float32)
        zeros = jnp.zeros((16,), jnp.float32)

        def astart(s):
            off = s * NVOX
            return pl.multiple_of((off >> 3) << 3, 8)

        def delta(s):
            return (s * NVOX) & 7

        def do_sample(s, vox, hist):
            d = delta(s)
            def zbody(i, c):
                hist[pl.ds(i * 16, 16)] = zeros
                return c
            lax.fori_loop(0, 256, zbody, 0, unroll=8)
            v = vox[pl.ds(0, 16)]
            idx = jnp.where(lane >= d, lbase + v, trash)
            plsc.addupdate_scatter(hist, [idx], ones)

            def body(i, c):
                vv = vox[pl.ds(i * 16, 16)]
                plsc.addupdate_scatter(hist, [lbase + vv], ones)
                return c
            lax.fori_loop(1, NIT - 1, body, 0, unroll=6)
            v = vox[pl.ds((NIT - 1) * 16, 16)]
            idx = jnp.where(lane <= d, lbase + v, trash)
            plsc.addupdate_scatter(hist, [idx], ones)

        pltpu.async_copy(vflat_hbm.at[pl.ds(astart(base), NPAD)], vox0, si0)

        def pair(k, c):
            s0 = base + 2 * k
            s1 = s0 + 1
            pltpu.async_copy(vflat_hbm.at[pl.ds(astart(s1), NPAD)], vox1, si1)
            pltpu.make_async_copy(
                vflat_hbm.at[pl.ds(astart(s0), NPAD)], vox0, si0).wait()

            @pl.when(k >= 1)
            def _():
                pltpu.make_async_copy(
                    hist0.at[pl.ds(0, 4096)], hout_hbm.at[s0 - 2], so0).wait()

            do_sample(s0, vox0, hist0)
            pltpu.async_copy(hist0.at[pl.ds(0, 4096)], hout_hbm.at[s0], so0)

            @pl.when(k + 1 < S // 2)
            def _():
                pltpu.async_copy(
                    vflat_hbm.at[pl.ds(astart(s0 + 2), NPAD)], vox0, si0)

            pltpu.make_async_copy(
                vflat_hbm.at[pl.ds(astart(s1), NPAD)], vox1, si1).wait()

            @pl.when(k >= 1)
            def _():
                pltpu.make_async_copy(
                    hist1.at[pl.ds(0, 4096)], hout_hbm.at[s1 - 2], so1).wait()

            do_sample(s1, vox1, hist1)
            pltpu.async_copy(hist1.at[pl.ds(0, 4096)], hout_hbm.at[s1], so1)
            return c

        lax.fori_loop(0, S // 2, pair, 0)
        pltpu.make_async_copy(
            hist0.at[pl.ds(0, 4096)], hout_hbm.at[base + S - 2], so0).wait()
        pltpu.make_async_copy(
            hist1.at[pl.ds(0, 4096)], hout_hbm.at[base + S - 1], so1).wait()

    return hist_kernel(vflat)


def _tbl_body(e2_ref, w2_ref, out_ref):
    out_ref[...] = jnp.dot(
        e2_ref[...].astype(jnp.float32), w2_ref[...].astype(jnp.float32),
        preferred_element_type=jnp.float32).astype(jnp.bfloat16)


def _make_tbl(embed2, W2):
    return pl.pallas_call(
        _tbl_body,
        grid=(125,),
        in_specs=[
            pl.BlockSpec((256, 16), lambda p: (0, 0)),
            pl.BlockSpec((16, 128), lambda p: (p, 0)),
        ],
        out_specs=pl.BlockSpec((256, 128), lambda p: (p, 0)),
        out_shape=jax.ShapeDtypeStruct((125 * 256, 128), jnp.bfloat16),
    )(embed2, W2)


def _spatial_body(cv_ref, rg_ref, bl_ref, tbl_ref, b2_ref, w3_ref, b3_ref,
                  w4b_ref, b4_ref, out_ref, oh_ref):
    cvv = cv_ref[...]
    iota = lax.broadcasted_iota(jnp.int32, (cvv.shape[0], 256), 1)
    for p in range(125):
        oh_ref[:, 256 * p:256 * (p + 1)] = (
            cvv[:, p:p + 1] == iota).astype(jnp.bfloat16)
    sp = jax.nn.relu(
        jnp.dot(oh_ref[...], tbl_ref[...],
                preferred_element_type=jnp.float32) + b2_ref[...])

    rgv = rg_ref[...]
    logc = jnp.sum(((rgv >= 8) & (rgv <= 10)).astype(jnp.float32),
                   axis=1, keepdims=True) * (1.0 / 125.0)
    orec = jnp.sum(((rgv >= 14) & (rgv <= 19)).astype(jnp.float32),
                   axis=1, keepdims=True) * (1.0 / 125.0)
    blv = bl_ref[...]
    sol = jnp.sum(((blv > 0) & (blv != 255)).astype(jnp.float32),
                  axis=1, keepdims=True) * (1.0 / 8.0)
    w3 = w3_ref[...]
    kf = jax.nn.relu(logc * w3[0:1, :] + orec * w3[1:2, :] + sol * w3[2:3, :]
                     + b3_ref[...])

    w4b = w4b_ref[...]
    out_ref[...] = (jnp.dot(sp, w4b[0:128], preferred_element_type=jnp.float32)
                    + jnp.dot(kf, w4b[128:160],
                              preferred_element_type=jnp.float32)
                    + b4_ref[...])


def _spatial(center, region, below, tbl, b2, W3, b3, W4b, b4):
    B = center.shape[0]
    BT = 256
    rep = lambda i: (0, 0)
    row = lambda i: (i, 0)
    return pl.pallas_call(
        _spatial_body,
        grid=(B // BT,),
        in_specs=[
            pl.BlockSpec((BT, 125), row),
            pl.BlockSpec((BT, 729), row),
            pl.BlockSpec((BT, 8), row),
            pl.BlockSpec((125 * 256, 128), rep),
            pl.BlockSpec((1, 128), rep),
            pl.BlockSpec((3, 32), rep),
            pl.BlockSpec((1, 32), rep),
            pl.BlockSpec((160, 256), rep),
            pl.BlockSpec((1, 256), rep),
        ],
        out_specs=pl.BlockSpec((BT, 256), row),
        out_shape=jax.ShapeDtypeStruct((B, 256), jnp.float32),
        scratch_shapes=[pltpu.VMEM((BT, 125 * 256), jnp.bfloat16)],
    )(center, region, below, tbl, b2, W3, b3, W4b, b4)


def _final_body(h_ref, e1s_ref, w1_ref, b1_ref, w4a_ref, part_ref, out_ref):
    mean = jnp.dot(h_ref[...], e1s_ref[...],
                   preferred_element_type=jnp.float32) * (1.0 / NVOX)
    hf = jax.nn.relu(jnp.dot(mean, w1_ref[...],
                             preferred_element_type=jnp.float32) + b1_ref[...])
    out_ref[...] = jax.nn.relu(
        jnp.dot(hf, w4a_ref[...], preferred_element_type=jnp.float32)
        + part_ref[...])


def _final(h, e1s, W1, b1, W4a, partial):
    B = h.shape[0]
    BT = 256
    rep = lambda i: (0, 0)
    row = lambda i: (i, 0)
    return pl.pallas_call(
        _final_body,
        grid=(B // BT,),
        in_specs=[
            pl.BlockSpec((BT, 4096), row),
            pl.BlockSpec((4096, 8), rep),
            pl.BlockSpec((8, 64), rep),
            pl.BlockSpec((1, 64), rep),
            pl.BlockSpec((64, 256), rep),
            pl.BlockSpec((BT, 256), row),
        ],
        out_specs=pl.BlockSpec((BT, 256), row),
        out_shape=jax.ShapeDtypeStruct((B, 256), jnp.float32),
    )(h, e1s, W1, b1, W4a, partial)


def kernel(voxels, embed1, W1, b1, embed2, W2, b2, W3, b3, W4, b4):
    B = voxels.shape[0]
    vflat = voxels.reshape(-1)
    center = voxels[:, 6:11, 6:11, 6:11].reshape(B, 125)
    region = voxels[:, 4:13, 4:13, 4:13].reshape(B, 729)
    below = voxels[:, 8, 0:8, 8]
    e1s = jnp.tile(embed1, (16, 1))

    h = _sc_histogram(vflat, B)
    tbl = _make_tbl(embed2, W2)
    partial = _spatial(center, region, below, tbl, b2.reshape(1, 128),
                       W3, b3.reshape(1, 32), W4[64:224], b4.reshape(1, 256))
    return _final(h, e1s, W1, b1.reshape(1, 64), W4[0:64], partial)
